# baseline (speedup 1.0000x reference)
import functools

import jax
import jax.numpy as jnp
from jax import lax
from jax.experimental import pallas as pl
from jax.experimental.pallas import tpu as pltpu
from jax.experimental.pallas import tpu_sc as plsc

B, T, D, F = 16, 512, 256, 256
M = 2048
PAD = 8
ZERO_IDX = B * T
NW = 32
B_PER_W = (B * M) // NW
CH = 128
NCHUNK = B_PER_W // CH


def _idx_body(t_ref, gidx_ref):
    b = pl.program_id(0)
    dur = t_ref[0].astype(jnp.float32)
    tt = lax.broadcasted_iota(jnp.int32, (T, T), 0)
    uu = lax.broadcasted_iota(jnp.int32, (T, T), 1)
    tri = (uu <= tt).astype(jnp.float32)
    cum = lax.dot_general(tri, dur, (((1,), (1,)), ((), ())),
                          preferred_element_type=jnp.float32)
    m_row = lax.broadcasted_iota(jnp.int32, (1, M), 1)
    cnt = jnp.sum((cum.astype(jnp.int32) <= m_row).astype(jnp.int32), axis=0,
                  keepdims=True)
    gidx_ref[0] = jnp.where(cnt < T, b * T + cnt, ZERO_IDX)


def _compute_gidx(target):
    t3 = target.reshape(B, 1, T)
    return pl.pallas_call(
        _idx_body,
        grid=(B,),
        in_specs=[pl.BlockSpec((1, 1, T), lambda b: (b, 0, 0))],
        out_specs=pl.BlockSpec((1, 1, M), lambda b: (b, 0, 0)),
        out_shape=jax.ShapeDtypeStruct((B, 1, M), jnp.int32),
    )(t3)


def _gather(table, gidx2):
    mesh = plsc.VectorSubcoreMesh(core_axis_name="c", subcore_axis_name="s")

    @functools.partial(
        pl.kernel,
        mesh=mesh,
        out_type=jax.ShapeDtypeStruct((B * M, D), jnp.float32),
        scratch_types=[
            pltpu.VMEM((NCHUNK, CH), jnp.int32),
            pltpu.VMEM((CH, D), jnp.float32),
            pltpu.VMEM((CH, D), jnp.float32),
            pltpu.SemaphoreType.DMA,
            pltpu.SemaphoreType.DMA,
        ],
    )
    def k(table_hbm, idx_hbm, out_hbm, idx_v, buf0, buf1, sem0, sem1):
        wid = lax.axis_index("s") * 2 + lax.axis_index("c")
        pltpu.sync_copy(idx_hbm.at[pl.ds(wid * NCHUNK, NCHUNK)], idx_v)
        bufs = (buf0, buf1)
        sems = (sem0, sem1)
        cps = [pltpu.async_copy(table_hbm.at[idx_v.at[0]], buf0, sem0), None]
        for c in range(NCHUNK):
            cur, nxt = c % 2, (c + 1) % 2
            if c + 1 < NCHUNK:
                cps[nxt] = pltpu.async_copy(
                    table_hbm.at[idx_v.at[c + 1]], bufs[nxt], sems[nxt])
            cps[cur].wait()
            pltpu.sync_copy(bufs[cur],
                            out_hbm.at[pl.ds(wid * B_PER_W + c * CH, CH)])

    return k(table, gidx2)


def _dp_body(x_ref, w1_ref, b1_ref, g1_ref, be1_ref, w2_ref, b2_ref, g2_ref,
             be2_ref, lw_ref, lb_ref, dp_ref):
    def conv_ln(h, w_ref, b_ref, g_ref, be_ref):
        row = lax.broadcasted_iota(jnp.int32, (T, 1), 0)
        hm1 = jnp.where(row == 0, 0.0, pltpu.roll(h, 1, 0))
        hp1 = jnp.where(row == T - 1, 0.0, pltpu.roll(h, T - 1, 0))
        hcat = jnp.concatenate([hm1, h, hp1], axis=1)
        y = jnp.dot(hcat, w_ref[...],
                    preferred_element_type=jnp.float32) + b_ref[...]
        y = jnp.maximum(y, 0.0)
        mu = jnp.mean(y, axis=1, keepdims=True)
        var = jnp.mean((y - mu) ** 2, axis=1, keepdims=True)
        return (y - mu) * lax.rsqrt(var + 1e-5) * g_ref[...] + be_ref[...]

    h = conv_ln(x_ref[0], w1_ref, b1_ref, g1_ref, be1_ref)
    h = conv_ln(h, w2_ref, b2_ref, g2_ref, be2_ref)
    dp = lax.dot_general(lw_ref[...], h, (((1,), (1,)), ((), ())),
                         preferred_element_type=jnp.float32)
    dp_ref[0] = dp + lb_ref[0, 0]


def _duration_predictor(x, w1c, b1, g1, be1, w2c, b2, g2, be2, lw, lb):
    full = lambda s: pl.BlockSpec(s, lambda b: tuple(0 for _ in s))
    dp3 = pl.pallas_call(
        _dp_body,
        grid=(B,),
        in_specs=[
            pl.BlockSpec((1, T, D), lambda b: (b, 0, 0)),
            full((3 * D, F)), full((1, F)), full((1, F)), full((1, F)),
            full((3 * F, F)), full((1, F)), full((1, F)), full((1, F)),
            full((1, F)), full((1, 1)),
        ],
        out_specs=pl.BlockSpec((1, 1, T), lambda b: (b, 0, 0)),
        out_shape=jax.ShapeDtypeStruct((B, 1, T), jnp.float32),
    )(x, w1c, b1, g1, be1, w2c, b2, g2, be2, lw, lb)
    return dp3.reshape(B, T)


def kernel(x, target, mel_max_length, conv1_w, conv1_b, ln1_g, ln1_b, conv2_w,
           conv2_b, ln2_g, ln2_b, lin_w, lin_b):
    w1c = conv1_w.transpose(2, 1, 0).reshape(3 * D, F)
    w2c = conv2_w.transpose(2, 1, 0).reshape(3 * F, F)
    b1 = conv1_b.reshape(1, F)
    b2 = conv2_b.reshape(1, F)
    g1, be1 = ln1_g.reshape(1, F), ln1_b.reshape(1, F)
    g2, be2 = ln2_g.reshape(1, F), ln2_b.reshape(1, F)
    lb = lin_b.reshape(1, 1)
    table = jnp.concatenate(
        [x.reshape(B * T, D), jnp.zeros((PAD, D), x.dtype)], axis=0)

    gidx = _compute_gidx(target)
    out_flat = _gather(table, gidx.reshape(NW * NCHUNK, CH))
    out = out_flat.reshape(B, M, D)
    dp = _duration_predictor(x, w1c, b1, g1, be1, w2c, b2, g2, be2, lin_w, lb)
    return (out, dp)


# speedup vs baseline: 1.0014x; 1.0014x over previous
import functools

import jax
import jax.numpy as jnp
from jax import lax
from jax.experimental import pallas as pl
from jax.experimental.pallas import tpu as pltpu
from jax.experimental.pallas import tpu_sc as plsc

B, T, D, F = 16, 512, 256, 256
M = 2048
PAD = 8
ZERO_IDX = B * T
NW = 32
B_PER_W = (B * M) // NW
CH = 128
NCHUNK = B_PER_W // CH


def _idx_body(t_ref, gidx_ref):
    b = pl.program_id(0)
    dur = t_ref[0].astype(jnp.float32)
    tt = lax.broadcasted_iota(jnp.int32, (T, T), 0)
    uu = lax.broadcasted_iota(jnp.int32, (T, T), 1)
    tri = (uu <= tt).astype(jnp.float32)
    cum = lax.dot_general(tri, dur, (((1,), (1,)), ((), ())),
                          preferred_element_type=jnp.float32)
    m_row = lax.broadcasted_iota(jnp.int32, (1, M), 1)
    cnt = jnp.sum((cum.astype(jnp.int32) <= m_row).astype(jnp.int32), axis=0,
                  keepdims=True)
    gidx_ref[0] = jnp.where(cnt < T, b * T + cnt, ZERO_IDX)


def _compute_gidx(target):
    t3 = target.reshape(B, 1, T)
    return pl.pallas_call(
        _idx_body,
        grid=(B,),
        in_specs=[pl.BlockSpec((1, 1, T), lambda b: (b, 0, 0))],
        out_specs=pl.BlockSpec((1, 1, M), lambda b: (b, 0, 0)),
        out_shape=jax.ShapeDtypeStruct((B, 1, M), jnp.int32),
    )(t3)


def _gather(table, gidx2):
    mesh = plsc.VectorSubcoreMesh(core_axis_name="c", subcore_axis_name="s")

    NB = 3

    @functools.partial(
        pl.kernel,
        mesh=mesh,
        out_type=jax.ShapeDtypeStruct((B * M, D), jnp.float32),
        scratch_types=[
            pltpu.VMEM((NCHUNK, CH), jnp.int32),
            pltpu.VMEM((NB, CH, D), jnp.float32),
        ] + [pltpu.SemaphoreType.DMA] * (2 * NB),
    )
    def k(table_hbm, idx_hbm, out_hbm, idx_v, bufs, *sems):
        gsems, osems = sems[:NB], sems[NB:]
        wid = lax.axis_index("s") * 2 + lax.axis_index("c")
        pltpu.sync_copy(idx_hbm.at[pl.ds(wid * NCHUNK, NCHUNK)], idx_v)

        def gather(c):
            return pltpu.async_copy(
                table_hbm.at[idx_v.at[c]], bufs.at[c % NB], gsems[c % NB])

        def put(c):
            return pltpu.async_copy(
                bufs.at[c % NB],
                out_hbm.at[pl.ds(wid * B_PER_W + c * CH, CH)], osems[c % NB])

        gcp = [None] * NCHUNK
        ocp = [None] * NCHUNK
        for c in range(NB):
            gcp[c] = gather(c)
        for c in range(NCHUNK):
            gcp[c].wait()
            ocp[c] = put(c)
            if c >= 1 and c + 2 < NCHUNK:
                ocp[c - 1].wait()
                gcp[c + 2] = gather(c + 2)
        ocp[NCHUNK - 2].wait()
        ocp[NCHUNK - 1].wait()

    return k(table, gidx2)


def _dp_body(x_ref, w1_ref, b1_ref, g1_ref, be1_ref, w2_ref, b2_ref, g2_ref,
             be2_ref, lw_ref, lb_ref, dp_ref):
    def conv_ln(h, w_ref, b_ref, g_ref, be_ref):
        row = lax.broadcasted_iota(jnp.int32, (T, 1), 0)
        hm1 = jnp.where(row == 0, 0.0, pltpu.roll(h, 1, 0))
        hp1 = jnp.where(row == T - 1, 0.0, pltpu.roll(h, T - 1, 0))
        hcat = jnp.concatenate([hm1, h, hp1], axis=1)
        y = jnp.dot(hcat, w_ref[...],
                    preferred_element_type=jnp.float32) + b_ref[...]
        y = jnp.maximum(y, 0.0)
        mu = jnp.mean(y, axis=1, keepdims=True)
        var = jnp.mean((y - mu) ** 2, axis=1, keepdims=True)
        return (y - mu) * lax.rsqrt(var + 1e-5) * g_ref[...] + be_ref[...]

    h = conv_ln(x_ref[0], w1_ref, b1_ref, g1_ref, be1_ref)
    h = conv_ln(h, w2_ref, b2_ref, g2_ref, be2_ref)
    dp = lax.dot_general(lw_ref[...], h, (((1,), (1,)), ((), ())),
                         preferred_element_type=jnp.float32)
    dp_ref[0] = dp + lb_ref[0, 0]


def _duration_predictor(x, w1c, b1, g1, be1, w2c, b2, g2, be2, lw, lb):
    full = lambda s: pl.BlockSpec(s, lambda b: tuple(0 for _ in s))
    dp3 = pl.pallas_call(
        _dp_body,
        grid=(B,),
        in_specs=[
            pl.BlockSpec((1, T, D), lambda b: (b, 0, 0)),
            full((3 * D, F)), full((1, F)), full((1, F)), full((1, F)),
            full((3 * F, F)), full((1, F)), full((1, F)), full((1, F)),
            full((1, F)), full((1, 1)),
        ],
        out_specs=pl.BlockSpec((1, 1, T), lambda b: (b, 0, 0)),
        out_shape=jax.ShapeDtypeStruct((B, 1, T), jnp.float32),
    )(x, w1c, b1, g1, be1, w2c, b2, g2, be2, lw, lb)
    return dp3.reshape(B, T)


def kernel(x, target, mel_max_length, conv1_w, conv1_b, ln1_g, ln1_b, conv2_w,
           conv2_b, ln2_g, ln2_b, lin_w, lin_b):
    w1c = conv1_w.transpose(2, 1, 0).reshape(3 * D, F)
    w2c = conv2_w.transpose(2, 1, 0).reshape(3 * F, F)
    b1 = conv1_b.reshape(1, F)
    b2 = conv2_b.reshape(1, F)
    g1, be1 = ln1_g.reshape(1, F), ln1_b.reshape(1, F)
    g2, be2 = ln2_g.reshape(1, F), ln2_b.reshape(1, F)
    lb = lin_b.reshape(1, 1)
    table = jnp.concatenate(
        [x.reshape(B * T, D), jnp.zeros((PAD, D), x.dtype)], axis=0)

    gidx = _compute_gidx(target)
    out_flat = _gather(table, gidx.reshape(NW * NCHUNK, CH))
    out = out_flat.reshape(B, M, D)
    dp = _duration_predictor(x, w1c, b1, g1, be1, w2c, b2, g2, be2, lin_w, lb)
    return (out, dp)
